# baseline (speedup 1.0000x reference)
import jax
import jax.numpy as jnp
from jax import lax
from jax.experimental import pallas as pl
from jax.experimental.pallas import tpu as pltpu
from jax.experimental.pallas import tpu_sc as plsc


def _lookup_body(idx_hbm, table_hbm, out_hbm, idx_s):
    c = lax.axis_index("c")

    @pl.when(c == 0)
    def _():
        pltpu.sync_copy(idx_hbm, idx_s)
        t = idx_s[0]
        pltpu.sync_copy(table_hbm.at[pl.ds(t, 1)], out_hbm)


def kernel(token, table):
    emb = table.shape[1]
    idx = jnp.asarray(token, jnp.int32).reshape((1,))
    mesh = plsc.ScalarSubcoreMesh(axis_name="c", num_cores=1)
    k = pl.kernel(
        _lookup_body,
        out_type=jax.ShapeDtypeStruct((1, emb), jnp.float32),
        mesh=mesh,
        scratch_types=[pltpu.SMEM((1,), jnp.int32)],
    )
    out = k(idx, table)
    return jnp.squeeze(out, axis=0)


# speedup vs baseline: 10.1316x; 10.1316x over previous
import jax
import jax.numpy as jnp
from jax.experimental import pallas as pl
from jax.experimental.pallas import tpu as pltpu


def _lookup_body(tok_ref, table_ref, out_ref, sem):
    t = tok_ref[0]
    pltpu.make_async_copy(table_ref.at[t], out_ref, sem).start()
    pltpu.make_async_copy(table_ref.at[t], out_ref, sem).wait()


def kernel(token, table):
    emb = table.shape[1]
    tok = jnp.asarray(token, jnp.int32).reshape((1,))
    return pl.pallas_call(
        _lookup_body,
        in_specs=[
            pl.BlockSpec(memory_space=pltpu.SMEM),
            pl.BlockSpec(memory_space=pl.ANY),
        ],
        out_specs=pl.BlockSpec(memory_space=pl.ANY),
        out_shape=jax.ShapeDtypeStruct((emb,), jnp.float32),
        scratch_shapes=[pltpu.SemaphoreType.DMA],
    )(tok, table)
